# Initial kernel scaffold; baseline (speedup 1.0000x reference)
import jax
import jax.numpy as jnp
from jax.experimental import pallas as pl


def kernel(probe_x, gallery_x, bn_weight, bn_bias, bn_mean, bn_var, W, b):
    raise NotImplementedError("write your pallas kernel here")


# speedup vs baseline: 2.5514x; 2.5514x over previous
import functools

import jax
import jax.numpy as jnp
from jax.experimental import pallas as pl
from jax.experimental.pallas import tpu as pltpu

FEAT = 128
NCLS = 2
EPS = 1e-5


def _cls_kernel(p_ref, g_ref, bw_ref, bb_ref, bm_ref, bv_ref, w_ref, b_ref,
                o0_ref, o1_ref):
    P = p_ref[...]
    G = g_ref[...]
    inv = bw_ref[...] * jax.lax.rsqrt(bv_ref[...] + EPS)
    shift = bb_ref[...] - bm_ref[...] * inv
    P2 = P * P
    G2 = G * G
    dims = (((1,), (1,)), ((), ()))
    for c, o_ref in ((0, o0_ref), (1, o1_ref)):
        v = inv * w_ref[c:c + 1, :]
        M = jax.lax.dot_general(P * v, G, dims,
                                preferred_element_type=jnp.float32,
                                precision=jax.lax.Precision.HIGHEST)
        A = jax.lax.dot_general(P2, v, dims,
                                preferred_element_type=jnp.float32,
                                precision=jax.lax.Precision.HIGHEST)
        B = jax.lax.dot_general(v, G2, dims,
                                preferred_element_type=jnp.float32,
                                precision=jax.lax.Precision.HIGHEST)
        C = jnp.sum(shift * w_ref[c:c + 1, :]) + b_ref[0, c]
        o_ref[...] = A + B - 2.0 * M + C


@functools.partial(jax.jit, static_argnames=("interpret",))
def kernel(probe_x, gallery_x, bn_weight, bn_bias, bn_mean, bn_var, W, b,
           interpret=False):
    NP, F = probe_x.shape
    NG = gallery_x.shape[0]
    BP = NP // 2

    row = lambda x: x.reshape(1, F)
    full = lambda shape: pl.BlockSpec(shape, lambda i: (0,) * len(shape))

    out0, out1 = pl.pallas_call(
        _cls_kernel,
        grid=(2,),
        in_specs=[
            pl.BlockSpec((BP, F), lambda i: (i, 0)),
            full((NG, F)),
            full((1, F)), full((1, F)), full((1, F)), full((1, F)),
            full((NCLS, F)),
            full((1, NCLS)),
        ],
        out_specs=[
            pl.BlockSpec((BP, NG), lambda i: (i, 0)),
            pl.BlockSpec((BP, NG), lambda i: (i, 0)),
        ],
        out_shape=[
            jax.ShapeDtypeStruct((NP, NG), jnp.float32),
            jax.ShapeDtypeStruct((NP, NG), jnp.float32),
        ],
        compiler_params=pltpu.CompilerParams(
            dimension_semantics=("parallel",)),
        interpret=interpret,
    )(probe_x, gallery_x, row(bn_weight), row(bn_bias), row(bn_mean),
      row(bn_var), W, b.reshape(1, NCLS))

    return jnp.stack([out0, out1], axis=-1)
